# Initial kernel scaffold; baseline (speedup 1.0000x reference)
import jax
import jax.numpy as jnp
from jax.experimental import pallas as pl


def kernel(node_features, edge_features, history_0, subg_norm, self_layer_nid, edge_src, edge_dst, gru_Wih, gru_Whh, gru_bih, gru_bhh, phi_W, phi_b, phi_g, phi_beta, out_W, out_b, out_g, out_beta, fc1_W, fc1_b, fc2_W, fc2_b):
    raise NotImplementedError("write your pallas kernel here")


# speedup vs baseline: 2.7138x; 2.7138x over previous
import functools

import jax
import jax.numpy as jnp
from jax import lax
from jax.experimental import pallas as pl
from jax.experimental.pallas import tpu as pltpu
from jax.experimental.pallas import tpu_sc as plsc

F32 = jnp.float32

N0 = 10000
N1 = 10000
E = 320000
EDGE_IN = 16
NODE_IN = 128
H = 64
FC = 128
C = 40

NC = 2
NS = 16
NW = NC * NS

EW = E // NW
ECH = 80
ENC = EW // ECH

NPAD = 10240
SCH = 80
SNC = NPAD // (NW * SCH)

N1P = 10240
ROWS_PT = N1P // NS

BN0 = 2000
BE = 2000
BN = 2000


def _ln_relu(x, g, b):
    m = jnp.mean(x, axis=-1, keepdims=True)
    xm = x - m
    v = jnp.mean(xm * xm, axis=-1, keepdims=True)
    return jnp.maximum(xm * lax.rsqrt(v + 1e-5) * g + b, 0.0)


def _nodepre_body(nf, hist, wa, wb, t_out, ps_out):
    pa = jnp.dot(nf[...], wa[...], preferred_element_type=F32)
    t_out[...] = jnp.concatenate([pa, hist[...]], axis=1)
    ps = jnp.dot(nf[...], wb[...], preferred_element_type=F32)
    ps_out[...] = jnp.concatenate([ps, ps], axis=1)


def _node_precompute(nf, hist, wsrc_t, wself_t):
    return pl.pallas_call(
        _nodepre_body,
        grid=(N0 // BN0,),
        in_specs=[
            pl.BlockSpec((BN0, NODE_IN), lambda i: (i, 0)),
            pl.BlockSpec((BN0, H), lambda i: (i, 0)),
            pl.BlockSpec((NODE_IN, H), lambda i: (0, 0)),
            pl.BlockSpec((NODE_IN, H), lambda i: (0, 0)),
        ],
        out_specs=[
            pl.BlockSpec((BN0, NODE_IN), lambda i: (i, 0)),
            pl.BlockSpec((BN0, NODE_IN), lambda i: (i, 0)),
        ],
        out_shape=[
            jax.ShapeDtypeStruct((N0, NODE_IN), F32),
            jax.ShapeDtypeStruct((N0, NODE_IN), F32),
        ],
    )(nf, hist, wsrc_t, wself_t)


def _gather_body(t_hbm, ps_hbm, esrc_hbm, snid_hbm, g_hbm, s_hbm,
                 eidx_v, erow_v, sidx_v, srow_v, sem):
    c = lax.axis_index("c")
    s = lax.axis_index("s")
    w = c * NS + s

    pltpu.sync_copy(esrc_hbm.at[w], eidx_v)

    def eloop(j, carry):
        pltpu.async_copy(t_hbm.at[eidx_v.at[j]], erow_v, sem).wait()
        pltpu.sync_copy(erow_v, g_hbm.at[pl.ds((w * ENC + j) * ECH, ECH)])
        return carry

    lax.fori_loop(0, ENC, eloop, 0)

    pltpu.sync_copy(snid_hbm.at[w], sidx_v)

    def sloop(j, carry):
        pltpu.async_copy(ps_hbm.at[sidx_v.at[j]], srow_v, sem).wait()
        pltpu.sync_copy(srow_v, s_hbm.at[pl.ds((w * SNC + j) * SCH, SCH)])
        return carry

    lax.fori_loop(0, SNC, sloop, 0)


def _sc_gather(table, ps, esrc3, snid3):
    mesh = plsc.VectorSubcoreMesh(core_axis_name="c", subcore_axis_name="s",
                                  num_cores=NC, num_subcores=NS)
    return pl.kernel(
        _gather_body,
        out_type=[
            jax.ShapeDtypeStruct((E, NODE_IN), F32),
            jax.ShapeDtypeStruct((NPAD, NODE_IN), F32),
        ],
        mesh=mesh,
        scratch_types=[
            pltpu.VMEM((ENC, ECH), jnp.int32),
            pltpu.VMEM((ECH, NODE_IN), F32),
            pltpu.VMEM((SNC, SCH), jnp.int32),
            pltpu.VMEM((SCH, NODE_IN), F32),
            pltpu.SemaphoreType.DMA,
        ],
    )(table, ps, esrc3, snid3)


def _edge_body(x0, x1, g,
               wihr, wihz, wihn, whhr, whhz, whhn, we,
               brz_r, brz_z, bihn, bhhn, phib, phig, phibeta,
               out):
    x0v = x0[...]
    x1v = x1[...]
    r1 = jax.nn.sigmoid(jnp.dot(x0v, wihr[...], preferred_element_type=F32)
                        + brz_r[...])
    z1 = jax.nn.sigmoid(jnp.dot(x0v, wihz[...], preferred_element_type=F32)
                        + brz_z[...])
    n1 = jnp.tanh(jnp.dot(x0v, wihn[...], preferred_element_type=F32)
                  + bihn[...] + r1 * bhhn[...])
    h1 = (1.0 - z1) * n1

    r2 = jax.nn.sigmoid(jnp.dot(x1v, wihr[...], preferred_element_type=F32)
                        + jnp.dot(h1, whhr[...], preferred_element_type=F32)
                        + brz_r[...])
    z2 = jax.nn.sigmoid(jnp.dot(x1v, wihz[...], preferred_element_type=F32)
                        + jnp.dot(h1, whhz[...], preferred_element_type=F32)
                        + brz_z[...])
    n2 = jnp.tanh(jnp.dot(x1v, wihn[...], preferred_element_type=F32)
                  + bihn[...]
                  + r2 * (jnp.dot(h1, whhn[...], preferred_element_type=F32)
                          + bhhn[...]))
    h2 = (1.0 - z2) * n2 + z2 * h1

    emb = 0.5 * (h1 + h2)
    gv = g[...]
    pre = (jnp.dot(emb, we[...], preferred_element_type=F32)
           + gv[:, :H] + phib[...])
    nb = _ln_relu(pre, phig[...], phibeta[...])
    delta = jnp.maximum(nb - gv[:, H:], 0.0)
    out[...] = jnp.concatenate([delta, delta], axis=1)


def _edge_stage(x0, x1, g, weights):
    wspecs = [pl.BlockSpec(w.shape, lambda i: tuple(0 for _ in w.shape))
              for w in weights]
    return pl.pallas_call(
        _edge_body,
        grid=(E // BE,),
        in_specs=[
            pl.BlockSpec((BE, EDGE_IN), lambda i: (i, 0)),
            pl.BlockSpec((BE, EDGE_IN), lambda i: (i, 0)),
            pl.BlockSpec((BE, NODE_IN), lambda i: (i, 0)),
        ] + wspecs,
        out_specs=pl.BlockSpec((BE, NODE_IN), lambda i: (i, 0)),
        out_shape=jax.ShapeDtypeStruct((E, NODE_IN), F32),
    )(x0, x1, g, *weights)


def _scatter_body(d_hbm, edst_hbm, z_hbm, p_hbm, acc_sh, idx_v, row_v):
    c = lax.axis_index("c")
    s = lax.axis_index("s")
    w = c * NS + s

    pltpu.sync_copy(z_hbm.at[pl.ds(s * ROWS_PT, ROWS_PT)],
                    acc_sh.at[pl.ds(s * ROWS_PT, ROWS_PT)])
    plsc.subcore_barrier()

    pltpu.sync_copy(edst_hbm.at[w], idx_v)

    def eloop(j, carry):
        pltpu.sync_copy(d_hbm.at[pl.ds((w * ENC + j) * ECH, ECH)], row_v)
        pltpu.sync_copy(row_v, acc_sh.at[idx_v.at[j]], add=True)
        return carry

    lax.fori_loop(0, ENC, eloop, 0)
    plsc.subcore_barrier()

    pltpu.sync_copy(acc_sh.at[pl.ds(s * ROWS_PT, ROWS_PT)],
                    p_hbm.at[c, pl.ds(s * ROWS_PT, ROWS_PT)])


def _sc_scatter(delta, edst3, zeros_init):
    mesh = plsc.VectorSubcoreMesh(core_axis_name="c", subcore_axis_name="s",
                                  num_cores=NC, num_subcores=NS)
    return pl.kernel(
        _scatter_body,
        out_type=jax.ShapeDtypeStruct((NC, N1P, NODE_IN), F32),
        mesh=mesh,
        scratch_types=[
            pltpu.VMEM_SHARED((N1P, NODE_IN), F32),
            pltpu.VMEM((ENC, ECH), jnp.int32),
            pltpu.VMEM((ECH, NODE_IN), F32),
        ],
    )(delta, edst3, zeros_init)


def _out_body(p, sg, subg,
              phib, phig, phibeta, owa, owb, outb, outg, outbeta,
              fc1w, fc1b, fc2w, fc2b,
              out):
    delta_nb = p[0, :, :H] + p[1, :, :H]
    self_h = _ln_relu(sg[:, :H] + phib[...], phig[...], phibeta[...])
    a = (delta_nb - self_h) * subg[...]
    pre = (jnp.dot(a, owa[...], preferred_element_type=F32)
           + jnp.dot(self_h, owb[...], preferred_element_type=F32)
           + outb[...])
    new_h = _ln_relu(pre, outg[...], outbeta[...])
    hf = jnp.dot(new_h, fc1w[...], preferred_element_type=F32) + fc1b[...]
    out[...] = jnp.dot(hf, fc2w[...], preferred_element_type=F32) + fc2b[...]


def _out_stage(partials, sgath, subg, weights):
    wspecs = [pl.BlockSpec(w.shape, lambda i: tuple(0 for _ in w.shape))
              for w in weights]
    return pl.pallas_call(
        _out_body,
        grid=(N1 // BN,),
        in_specs=[
            pl.BlockSpec((NC, BN, NODE_IN), lambda i: (0, i, 0)),
            pl.BlockSpec((BN, NODE_IN), lambda i: (i, 0)),
            pl.BlockSpec((BN, 1), lambda i: (i, 0)),
        ] + wspecs,
        out_specs=pl.BlockSpec((BN, C), lambda i: (i, 0)),
        out_shape=jax.ShapeDtypeStruct((N1, C), F32),
    )(partials, sgath, subg, *weights)


def kernel(node_features, edge_features, history_0, subg_norm,
           self_layer_nid, edge_src, edge_dst,
           gru_Wih, gru_Whh, gru_bih, gru_bhh,
           phi_W, phi_b, phi_g, phi_beta,
           out_W, out_b, out_g, out_beta,
           fc1_W, fc1_b, fc2_W, fc2_b):
    wsrc_t = phi_W[:, :NODE_IN].T
    wself_t = phi_W[:, H:].T
    we_t = phi_W[:, NODE_IN:].T
    wihr = gru_Wih[:H, :].T
    wihz = gru_Wih[H:2 * H, :].T
    wihn = gru_Wih[2 * H:, :].T
    whhr = gru_Whh[:H, :].T
    whhz = gru_Whh[H:2 * H, :].T
    whhn = gru_Whh[2 * H:, :].T
    brz_r = (gru_bih[:H] + gru_bhh[:H]).reshape(1, H)
    brz_z = (gru_bih[H:2 * H] + gru_bhh[H:2 * H]).reshape(1, H)
    bihn = gru_bih[2 * H:].reshape(1, H)
    bhhn = gru_bhh[2 * H:].reshape(1, H)
    phib = phi_b.reshape(1, H)
    phig = phi_g.reshape(1, H)
    phibeta = phi_beta.reshape(1, H)
    owa = out_W[:, :H].T
    owb = out_W[:, H:].T
    outb = out_b.reshape(1, H)
    outg = out_g.reshape(1, H)
    outbeta = out_beta.reshape(1, H)
    fc1t = fc1_W.T
    fc1b = fc1_b.reshape(1, FC)
    fc2t = fc2_W.T
    fc2b = fc2_b.reshape(1, C)

    esrc3 = edge_src.reshape(NW, ENC, ECH)
    edst3 = edge_dst.reshape(NW, ENC, ECH)
    snid_pad = jnp.concatenate(
        [self_layer_nid, jnp.zeros((NPAD - N1,), jnp.int32)]
    ).reshape(NW, SNC, SCH)
    x0 = edge_features[:, 0, :]
    x1 = edge_features[:, 1, :]
    zeros_init = jnp.zeros((N1P, NODE_IN), F32)

    table, ps = _node_precompute(node_features, history_0, wsrc_t, wself_t)
    g, s_pad = _sc_gather(table, ps, esrc3, snid_pad)
    delta = _edge_stage(x0, x1, g, [
        wihr, wihz, wihn, whhr, whhz, whhn, we_t,
        brz_r, brz_z, bihn, bhhn, phib, phig, phibeta,
    ])
    partials = _sc_scatter(delta, edst3, zeros_init)
    logit = _out_stage(partials, s_pad[:N1], subg_norm, [
        phib, phig, phibeta, owa, owb, outb, outg, outbeta,
        fc1t, fc1b, fc2t, fc2b,
    ])
    return logit


# speedup vs baseline: 2.8538x; 1.0516x over previous
import functools

import jax
import jax.numpy as jnp
from jax import lax
from jax.experimental import pallas as pl
from jax.experimental.pallas import tpu as pltpu
from jax.experimental.pallas import tpu_sc as plsc

F32 = jnp.float32

N0 = 10000
N1 = 10000
E = 320000
EDGE_IN = 16
NODE_IN = 128
H = 64
FC = 128
C = 40

NC = 2
NS = 16
NW = NC * NS

EW = E // NW
ECH = 80
ENC = EW // ECH

NPAD = 10240
SCH = 80
SNC = NPAD // (NW * SCH)

N1P = 10240
ROWS_PT = N1P // NS

BN0 = 2000
BE = 2000
BN = 2000


def _sigmoid(x):
    return 0.5 + 0.5 * jnp.tanh(0.5 * x)


def _ln_relu(x, g, b):
    m = jnp.mean(x, axis=-1, keepdims=True)
    xm = x - m
    v = jnp.mean(xm * xm, axis=-1, keepdims=True)
    return jnp.maximum(xm * lax.rsqrt(v + 1e-5) * g + b, 0.0)


def _nodepre_body(nf, hist, wa, wb, t_out, ps_out):
    pa = jnp.dot(nf[...], wa[...], preferred_element_type=F32)
    t_out[...] = jnp.concatenate([pa, hist[...]], axis=1)
    ps = jnp.dot(nf[...], wb[...], preferred_element_type=F32)
    ps_out[...] = jnp.concatenate([ps, ps], axis=1)


def _node_precompute(nf, hist, wsrc_t, wself_t):
    return pl.pallas_call(
        _nodepre_body,
        grid=(N0 // BN0,),
        in_specs=[
            pl.BlockSpec((BN0, NODE_IN), lambda i: (i, 0)),
            pl.BlockSpec((BN0, H), lambda i: (i, 0)),
            pl.BlockSpec((NODE_IN, H), lambda i: (0, 0)),
            pl.BlockSpec((NODE_IN, H), lambda i: (0, 0)),
        ],
        out_specs=[
            pl.BlockSpec((BN0, NODE_IN), lambda i: (i, 0)),
            pl.BlockSpec((BN0, NODE_IN), lambda i: (i, 0)),
        ],
        out_shape=[
            jax.ShapeDtypeStruct((N0, NODE_IN), F32),
            jax.ShapeDtypeStruct((N0, NODE_IN), F32),
        ],
    )(nf, hist, wsrc_t, wself_t)


def _gather_body(t_hbm, ps_hbm, esrc_hbm, snid_hbm, g_hbm, s_hbm,
                 eidx_v, erow_v, sidx_v, srow_v, sem):
    c = lax.axis_index("c")
    s = lax.axis_index("s")
    w = c * NS + s

    pltpu.sync_copy(esrc_hbm.at[w], eidx_v)

    def eloop(j, carry):
        pltpu.async_copy(t_hbm.at[eidx_v.at[j]], erow_v, sem).wait()
        pltpu.sync_copy(erow_v, g_hbm.at[pl.ds((w * ENC + j) * ECH, ECH)])
        return carry

    lax.fori_loop(0, ENC, eloop, 0)

    pltpu.sync_copy(snid_hbm.at[w], sidx_v)

    def sloop(j, carry):
        pltpu.async_copy(ps_hbm.at[sidx_v.at[j]], srow_v, sem).wait()
        pltpu.sync_copy(srow_v, s_hbm.at[pl.ds((w * SNC + j) * SCH, SCH)])
        return carry

    lax.fori_loop(0, SNC, sloop, 0)


def _sc_gather(table, ps, esrc3, snid3):
    mesh = plsc.VectorSubcoreMesh(core_axis_name="c", subcore_axis_name="s",
                                  num_cores=NC, num_subcores=NS)
    return pl.kernel(
        _gather_body,
        out_type=[
            jax.ShapeDtypeStruct((E, NODE_IN), F32),
            jax.ShapeDtypeStruct((NPAD, NODE_IN), F32),
        ],
        mesh=mesh,
        scratch_types=[
            pltpu.VMEM((ENC, ECH), jnp.int32),
            pltpu.VMEM((ECH, NODE_IN), F32),
            pltpu.VMEM((SNC, SCH), jnp.int32),
            pltpu.VMEM((SCH, NODE_IN), F32),
            pltpu.SemaphoreType.DMA,
        ],
    )(table, ps, esrc3, snid3)


def _edge_body(ef, g,
               wihr, wihz, wihn, whhr, whhz, whhn, we,
               brz_r, brz_z, bihn, bhhn, phib, phig, phibeta,
               out):
    efv = ef[...]
    x0v = efv[:, :EDGE_IN]
    x1v = efv[:, EDGE_IN:]
    r1 = _sigmoid(jnp.dot(x0v, wihr[...], preferred_element_type=F32)
                  + brz_r[...])
    z1 = _sigmoid(jnp.dot(x0v, wihz[...], preferred_element_type=F32)
                  + brz_z[...])
    n1 = jnp.tanh(jnp.dot(x0v, wihn[...], preferred_element_type=F32)
                  + bihn[...] + r1 * bhhn[...])
    h1 = (1.0 - z1) * n1

    r2 = _sigmoid(jnp.dot(x1v, wihr[...], preferred_element_type=F32)
                  + jnp.dot(h1, whhr[...], preferred_element_type=F32)
                  + brz_r[...])
    z2 = _sigmoid(jnp.dot(x1v, wihz[...], preferred_element_type=F32)
                  + jnp.dot(h1, whhz[...], preferred_element_type=F32)
                  + brz_z[...])
    n2 = jnp.tanh(jnp.dot(x1v, wihn[...], preferred_element_type=F32)
                  + bihn[...]
                  + r2 * (jnp.dot(h1, whhn[...], preferred_element_type=F32)
                          + bhhn[...]))
    h2 = (1.0 - z2) * n2 + z2 * h1

    emb = 0.5 * (h1 + h2)
    gv = g[...]
    pre = (jnp.dot(emb, we[...], preferred_element_type=F32)
           + gv[:, :H] + phib[...])
    nb = _ln_relu(pre, phig[...], phibeta[...])
    delta = jnp.maximum(nb - gv[:, H:], 0.0)
    out[...] = jnp.concatenate([delta, delta], axis=1)


def _edge_stage(ef, g, weights):
    wspecs = [pl.BlockSpec(w.shape, lambda i: tuple(0 for _ in w.shape))
              for w in weights]
    return pl.pallas_call(
        _edge_body,
        grid=(E // BE,),
        in_specs=[
            pl.BlockSpec((BE, 2 * EDGE_IN), lambda i: (i, 0)),
            pl.BlockSpec((BE, NODE_IN), lambda i: (i, 0)),
        ] + wspecs,
        out_specs=pl.BlockSpec((BE, NODE_IN), lambda i: (i, 0)),
        out_shape=jax.ShapeDtypeStruct((E, NODE_IN), F32),
    )(ef, g, *weights)


def _scatter_body(d_hbm, edst_hbm, z_hbm, p_hbm, acc_sh, idx_v, row_v):
    c = lax.axis_index("c")
    s = lax.axis_index("s")
    w = c * NS + s

    pltpu.sync_copy(z_hbm.at[pl.ds(s * ROWS_PT, ROWS_PT)],
                    acc_sh.at[pl.ds(s * ROWS_PT, ROWS_PT)])
    plsc.subcore_barrier()

    pltpu.sync_copy(edst_hbm.at[w], idx_v)

    def eloop(j, carry):
        pltpu.sync_copy(d_hbm.at[pl.ds((w * ENC + j) * ECH, ECH)], row_v)
        pltpu.sync_copy(row_v, acc_sh.at[idx_v.at[j]], add=True)
        return carry

    lax.fori_loop(0, ENC, eloop, 0)
    plsc.subcore_barrier()

    pltpu.sync_copy(acc_sh.at[pl.ds(s * ROWS_PT, ROWS_PT)],
                    p_hbm.at[c, pl.ds(s * ROWS_PT, ROWS_PT)])


def _sc_scatter(delta, edst3, zeros_init):
    mesh = plsc.VectorSubcoreMesh(core_axis_name="c", subcore_axis_name="s",
                                  num_cores=NC, num_subcores=NS)
    return pl.kernel(
        _scatter_body,
        out_type=jax.ShapeDtypeStruct((NC, N1P, NODE_IN), F32),
        mesh=mesh,
        scratch_types=[
            pltpu.VMEM_SHARED((N1P, NODE_IN), F32),
            pltpu.VMEM((ENC, ECH), jnp.int32),
            pltpu.VMEM((ECH, NODE_IN), F32),
        ],
    )(delta, edst3, zeros_init)


def _out_body(p, sg, subg,
              phib, phig, phibeta, owa, owb, outb, outg, outbeta,
              fc1w, fc1b, fc2w, fc2b,
              out):
    delta_nb = p[0, :, :H] + p[1, :, :H]
    self_h = _ln_relu(sg[:, :H] + phib[...], phig[...], phibeta[...])
    a = (delta_nb - self_h) * subg[...]
    pre = (jnp.dot(a, owa[...], preferred_element_type=F32)
           + jnp.dot(self_h, owb[...], preferred_element_type=F32)
           + outb[...])
    new_h = _ln_relu(pre, outg[...], outbeta[...])
    hf = jnp.dot(new_h, fc1w[...], preferred_element_type=F32) + fc1b[...]
    out[...] = jnp.dot(hf, fc2w[...], preferred_element_type=F32) + fc2b[...]


def _out_stage(partials, sgath, subg, weights):
    wspecs = [pl.BlockSpec(w.shape, lambda i: tuple(0 for _ in w.shape))
              for w in weights]
    return pl.pallas_call(
        _out_body,
        grid=(N1 // BN,),
        in_specs=[
            pl.BlockSpec((NC, BN, NODE_IN), lambda i: (0, i, 0)),
            pl.BlockSpec((BN, NODE_IN), lambda i: (i, 0)),
            pl.BlockSpec((BN, 1), lambda i: (i, 0)),
        ] + wspecs,
        out_specs=pl.BlockSpec((BN, C), lambda i: (i, 0)),
        out_shape=jax.ShapeDtypeStruct((N1, C), F32),
    )(partials, sgath, subg, *weights)


def kernel(node_features, edge_features, history_0, subg_norm,
           self_layer_nid, edge_src, edge_dst,
           gru_Wih, gru_Whh, gru_bih, gru_bhh,
           phi_W, phi_b, phi_g, phi_beta,
           out_W, out_b, out_g, out_beta,
           fc1_W, fc1_b, fc2_W, fc2_b):
    wsrc_t = phi_W[:, :NODE_IN].T
    wself_t = phi_W[:, H:].T
    we_t = phi_W[:, NODE_IN:].T
    wihr = gru_Wih[:H, :].T
    wihz = gru_Wih[H:2 * H, :].T
    wihn = gru_Wih[2 * H:, :].T
    whhr = gru_Whh[:H, :].T
    whhz = gru_Whh[H:2 * H, :].T
    whhn = gru_Whh[2 * H:, :].T
    brz_r = (gru_bih[:H] + gru_bhh[:H]).reshape(1, H)
    brz_z = (gru_bih[H:2 * H] + gru_bhh[H:2 * H]).reshape(1, H)
    bihn = gru_bih[2 * H:].reshape(1, H)
    bhhn = gru_bhh[2 * H:].reshape(1, H)
    phib = phi_b.reshape(1, H)
    phig = phi_g.reshape(1, H)
    phibeta = phi_beta.reshape(1, H)
    owa = out_W[:, :H].T
    owb = out_W[:, H:].T
    outb = out_b.reshape(1, H)
    outg = out_g.reshape(1, H)
    outbeta = out_beta.reshape(1, H)
    fc1t = fc1_W.T
    fc1b = fc1_b.reshape(1, FC)
    fc2t = fc2_W.T
    fc2b = fc2_b.reshape(1, C)

    esrc3 = edge_src.reshape(NW, ENC, ECH)
    edst3 = edge_dst.reshape(NW, ENC, ECH)
    snid_pad = jnp.concatenate(
        [self_layer_nid, jnp.zeros((NPAD - N1,), jnp.int32)]
    ).reshape(NW, SNC, SCH)
    ef2 = edge_features.reshape(E, 2 * EDGE_IN)
    zeros_init = jnp.zeros((N1P, NODE_IN), F32)

    table, ps = _node_precompute(node_features, history_0, wsrc_t, wself_t)
    g, s_pad = _sc_gather(table, ps, esrc3, snid_pad)
    delta = _edge_stage(ef2, g, [
        wihr, wihz, wihn, whhr, whhz, whhn, we_t,
        brz_r, brz_z, bihn, bhhn, phib, phig, phibeta,
    ])
    partials = _sc_scatter(delta, edst3, zeros_init)
    logit = _out_stage(partials, s_pad[:N1], subg_norm, [
        phib, phig, phibeta, owa, owb, outb, outg, outbeta,
        fc1t, fc1b, fc2t, fc2b,
    ])
    return logit
